# baseline (speedup 1.0000x reference)
import functools

import jax
import jax.numpy as jnp
from jax import lax
from jax.experimental import pallas as pl
from jax.experimental.pallas import tpu as pltpu
from jax.experimental.pallas import tpu_sc as plsc

_NBUF = 8
_LOOKAHEAD = 4


@functools.partial(jax.jit, static_argnames=("num_rows", "d", "ch", "nch"))
def _sc_gather(idx, table, *, num_rows, d, ch, nch):
    nw = 32
    bpw = num_rows // nw

    mesh = plsc.VectorSubcoreMesh(core_axis_name="c", subcore_axis_name="s")

    @functools.partial(
        pl.kernel,
        mesh=mesh,
        out_type=jax.ShapeDtypeStruct((num_rows, d), jnp.float32),
        scratch_types=[
            pltpu.VMEM((bpw,), jnp.int32),
            [pltpu.VMEM((ch, d), jnp.float32) for _ in range(_NBUF)],
            [pltpu.SemaphoreType.DMA for _ in range(_NBUF)],
            [pltpu.SemaphoreType.DMA for _ in range(_NBUF)],
        ],
        compiler_params=pltpu.CompilerParams(use_tc_tiling_on_sc=False),
    )
    def body(tokens_hbm, table_hbm, out_hbm, idx_v, bufs, gsems, wsems):
        wid = lax.axis_index("s") * 2 + lax.axis_index("c")
        row0 = wid * bpw

        def fire_gather(g, b):
            pltpu.async_copy(
                table_hbm.at[idx_v.at[pl.ds(g * ch, ch)]], bufs[b], gsems[b]
            )

        def drain_gather(b):
            pltpu.make_async_copy(
                out_hbm.at[pl.ds(0, ch)], bufs[b], gsems[b]
            ).wait()

        def fire_write(g, b):
            pltpu.async_copy(
                bufs[b], out_hbm.at[pl.ds(row0 + g * ch, ch)], wsems[b]
            )

        def drain_write(b):
            pltpu.make_async_copy(
                bufs[b], out_hbm.at[pl.ds(0, ch)], wsems[b]
            ).wait()

        pltpu.sync_copy(tokens_hbm.at[pl.ds(row0, bpw)], idx_v)

        for g in range(_NBUF):
            fire_gather(g, g)
            if g >= _LOOKAHEAD:
                drain_gather(g - _LOOKAHEAD)
                fire_write(g - _LOOKAHEAD, g - _LOOKAHEAD)

        def step(g0, carry):
            for i in range(_NBUF):
                g = g0 + i
                drain_write(i)
                fire_gather(g, i)
                bw = (i + _NBUF - _LOOKAHEAD) % _NBUF
                drain_gather(bw)
                fire_write(g - _LOOKAHEAD, bw)
            return carry

        lax.fori_loop(
            0, (nch - _NBUF) // _NBUF,
            lambda p, c: step(_NBUF + p * _NBUF, c), 0,
        )

        for g in range(nch, nch + _LOOKAHEAD):
            b = (g - _LOOKAHEAD) % _NBUF
            drain_gather(b)
            fire_write(g - _LOOKAHEAD, b)
        for b in range(_NBUF):
            drain_write(b)

    return body(idx, table)


def kernel(tokens, table):
    b, t = tokens.shape
    num_rows = b * t
    d = table.shape[1]
    idx = tokens.astype(jnp.int32).reshape(num_rows)
    ch = 200
    nch = num_rows // 32 // ch
    out = _sc_gather(idx, table, num_rows=num_rows, d=d, ch=ch, nch=nch)
    return out.reshape(b, t, d)


# speedup vs baseline: 1.3388x; 1.3388x over previous
import functools

import jax
import jax.numpy as jnp
from jax import lax
from jax.experimental import pallas as pl
from jax.experimental.pallas import tpu as pltpu
from jax.experimental.pallas import tpu_sc as plsc

_NBUF = 8
_LOOKAHEAD = 4


@functools.partial(jax.jit, static_argnames=("num_rows", "d", "ch", "nch"))
def _sc_gather(idx, table, *, num_rows, d, ch, nch):
    nw = 32
    bpw = num_rows // nw

    mesh = plsc.VectorSubcoreMesh(core_axis_name="c", subcore_axis_name="s")

    @functools.partial(
        pl.kernel,
        mesh=mesh,
        out_type=jax.ShapeDtypeStruct((num_rows, d), jnp.float32),
        scratch_types=[
            pltpu.VMEM((bpw,), jnp.int32),
            [pltpu.VMEM((ch, d), jnp.float32) for _ in range(_NBUF)],
            [pltpu.SemaphoreType.DMA for _ in range(_NBUF)],
            [pltpu.SemaphoreType.DMA for _ in range(_NBUF)],
        ],
        compiler_params=pltpu.CompilerParams(use_tc_tiling_on_sc=False),
    )
    def body(tokens_hbm, table_hbm, out_hbm, idx_v, bufs, gsems, wsems):
        wid = lax.axis_index("s") * 2 + lax.axis_index("c")
        row0 = wid * bpw

        def fire_gather(g, b):
            pltpu.async_copy(
                table_hbm.at[idx_v.at[pl.ds(g * ch, ch)]], bufs[b], gsems[b]
            )

        def drain_gather(b):
            pltpu.make_async_copy(
                out_hbm.at[pl.ds(0, ch)], bufs[b], gsems[b]
            ).wait()

        def fire_write(g, b):
            pltpu.async_copy(
                bufs[b], out_hbm.at[pl.ds(row0 + g * ch, ch)], wsems[b]
            )

        def drain_write(b):
            pltpu.make_async_copy(
                bufs[b], out_hbm.at[pl.ds(0, ch)], wsems[b]
            ).wait()

        pltpu.sync_copy(tokens_hbm.at[pl.ds(row0, bpw)], idx_v)

        for g in range(_NBUF):
            fire_gather(g, g)
            if g >= _LOOKAHEAD:
                drain_gather(g - _LOOKAHEAD)
                fire_write(g - _LOOKAHEAD, g - _LOOKAHEAD)

        def step(g0, carry):
            for i in range(_NBUF):
                g = g0 + i
                drain_write(i)
                fire_gather(g, i)
                bw = (i + _NBUF - _LOOKAHEAD) % _NBUF
                drain_gather(bw)
                fire_write(g - _LOOKAHEAD, bw)
            return carry

        lax.fori_loop(
            0, (nch - _NBUF) // _NBUF,
            lambda p, c: step(_NBUF + p * _NBUF, c), 0,
        )

        for g in range(nch, nch + _LOOKAHEAD):
            b = (g - _LOOKAHEAD) % _NBUF
            drain_gather(b)
            fire_write(g - _LOOKAHEAD, b)
        for b in range(_NBUF):
            drain_write(b)

    return body(idx, table)


_TC_BLK = 2048
_SPLIT = 245 * _TC_BLK


def _tc_format(tT):
    d, n = tT.shape
    nblk = (n + _TC_BLK - 1) // _TC_BLK - 1

    def body(lo_ref, hi_ref, o_ref):
        xcat = jnp.concatenate([lo_ref[...], hi_ref[...]], axis=0)
        row = jax.lax.broadcasted_iota(jnp.int32, (2 * d, 2 * d), 0)
        col = jax.lax.broadcasted_iota(jnp.int32, (2 * d, 2 * d), 1)
        ident = (row == col).astype(jnp.float32)
        o_ref[...] = jax.lax.dot_general(
            xcat, ident, (((0,), (0,)), ((), ())),
            preferred_element_type=jnp.float32,
        )

    return pl.pallas_call(
        body,
        grid=(_SPLIT // _TC_BLK,),
        in_specs=[
            pl.BlockSpec((d, _TC_BLK), lambda i: (0, i)),
            pl.BlockSpec(
                (d, _TC_BLK),
                lambda i: (0, jnp.minimum(i + _SPLIT // _TC_BLK, nblk)),
            ),
        ],
        out_specs=pl.BlockSpec((_TC_BLK, 2 * d), lambda i: (i, 0)),
        out_shape=jax.ShapeDtypeStruct((_SPLIT, 2 * d), jnp.float32),
    )(tT, tT)


def kernel(tokens, table):
    b, t = tokens.shape
    num_rows = b * t
    d = table.shape[1]
    idx = tokens.astype(jnp.int32).reshape(num_rows)
    idx2 = jnp.where(idx < _SPLIT, 2 * idx, 2 * (idx - _SPLIT) + 1)
    packed = _tc_format(table.T)
    tlin = packed.reshape(2 * _SPLIT, d)
    ch = 200
    nch = num_rows // 32 // ch
    out = _sc_gather(idx2, tlin, num_rows=num_rows, d=d, ch=ch, nch=nch)
    return out.reshape(b, t, d)


# speedup vs baseline: 2.0026x; 1.4958x over previous
import functools

import jax
import jax.numpy as jnp
from jax import lax
from jax.experimental import pallas as pl
from jax.experimental.pallas import tpu as pltpu
from jax.experimental.pallas import tpu_sc as plsc

_NBUF = 8
_LOOKAHEAD = 4


@functools.partial(jax.jit, static_argnames=("num_rows", "d", "ch", "nch"))
def _sc_gather(idx, table, *, num_rows, d, ch, nch):
    nw = 32
    bpw = num_rows // nw

    mesh = plsc.VectorSubcoreMesh(core_axis_name="c", subcore_axis_name="s")

    @functools.partial(
        pl.kernel,
        mesh=mesh,
        out_type=jax.ShapeDtypeStruct((num_rows, 2 * d), jnp.float32),
        scratch_types=[
            pltpu.VMEM((bpw,), jnp.int32),
            [pltpu.VMEM((ch, d), jnp.float32) for _ in range(_NBUF)],
            [pltpu.SemaphoreType.DMA for _ in range(_NBUF)],
            [pltpu.SemaphoreType.DMA for _ in range(_NBUF)],
        ],
        compiler_params=pltpu.CompilerParams(use_tc_tiling_on_sc=False),
    )
    def body(tokens_hbm, table_hbm, out_hbm, idx_v, bufs, gsems, wsems):
        wid = lax.axis_index("s") * 2 + lax.axis_index("c")
        row0 = wid * bpw

        def fire_gather(g, b):
            pltpu.async_copy(
                table_hbm.at[idx_v.at[pl.ds(g * ch, ch)]], bufs[b], gsems[b]
            )

        def drain_gather(b):
            pltpu.make_async_copy(
                out_hbm.at[pl.ds(0, ch)], bufs[b], gsems[b]
            ).wait()

        def fire_write(g, b):
            pltpu.async_copy(
                bufs[b],
                out_hbm.at[pl.ds(row0 + g * ch, ch), pl.ds(0, d)],
                wsems[b],
            )

        def drain_write(b):
            pltpu.make_async_copy(
                bufs[b], out_hbm.at[pl.ds(0, ch), pl.ds(0, d)], wsems[b]
            ).wait()

        pltpu.sync_copy(tokens_hbm.at[pl.ds(row0, bpw)], idx_v)

        for g in range(_NBUF):
            fire_gather(g, g)
            if g >= _LOOKAHEAD:
                drain_gather(g - _LOOKAHEAD)
                fire_write(g - _LOOKAHEAD, g - _LOOKAHEAD)

        def step(g0, carry):
            for i in range(_NBUF):
                g = g0 + i
                drain_write(i)
                fire_gather(g, i)
                bw = (i + _NBUF - _LOOKAHEAD) % _NBUF
                drain_gather(bw)
                fire_write(g - _LOOKAHEAD, bw)
            return carry

        lax.fori_loop(
            0, (nch - _NBUF) // _NBUF,
            lambda p, c: step(_NBUF + p * _NBUF, c), 0,
        )

        for g in range(nch, nch + _LOOKAHEAD):
            b = (g - _LOOKAHEAD) % _NBUF
            drain_gather(b)
            fire_write(g - _LOOKAHEAD, b)
        for b in range(_NBUF):
            drain_write(b)

    return body(idx, table)


_TC_BLK = 2048
_SPLIT = 245 * _TC_BLK


def _tc_format(tT):
    d, n = tT.shape
    nblk = (n + _TC_BLK - 1) // _TC_BLK - 1

    def body(lo_ref, hi_ref, o_ref):
        xcat = jnp.concatenate([lo_ref[...], hi_ref[...]], axis=0)
        row = jax.lax.broadcasted_iota(jnp.int32, (2 * d, 2 * d), 0)
        col = jax.lax.broadcasted_iota(jnp.int32, (2 * d, 2 * d), 1)
        ident = (row == col).astype(jnp.float32)
        o_ref[...] = jax.lax.dot_general(
            xcat, ident, (((0,), (0,)), ((), ())),
            preferred_element_type=jnp.float32,
        )

    return pl.pallas_call(
        body,
        grid=(_SPLIT // _TC_BLK,),
        in_specs=[
            pl.BlockSpec((d, _TC_BLK), lambda i: (0, i)),
            pl.BlockSpec(
                (d, _TC_BLK),
                lambda i: (0, jnp.minimum(i + _SPLIT // _TC_BLK, nblk)),
            ),
        ],
        out_specs=pl.BlockSpec((_TC_BLK, 2 * d), lambda i: (i, 0)),
        out_shape=jax.ShapeDtypeStruct((_SPLIT, 2 * d), jnp.float32),
    )(tT, tT)


def kernel(tokens, table):
    b, t = tokens.shape
    num_rows = b * t
    d = table.shape[1]
    idx = tokens.astype(jnp.int32).reshape(num_rows)
    idx2 = jnp.where(idx < _SPLIT, 2 * idx, 2 * (idx - _SPLIT) + 1)
    packed = _tc_format(table.T)
    tlin = packed.reshape(2 * _SPLIT, d)
    ch = 200
    nch = num_rows // 32 // ch
    out = _sc_gather(idx2, tlin, num_rows=num_rows, d=d, ch=ch, nch=nch)
    return out.reshape(b, t, 2 * d)[:, :, :d]
